# baseline (speedup 1.0000x reference)
import functools

import jax
import jax.numpy as jnp
from jax import lax
from jax.experimental import pallas as pl
from jax.experimental.pallas import tpu as pltpu
from jax.experimental.pallas import tpu_sc as plsc

NC = 2
NS = 16
CH = 128


@functools.partial(jax.jit, static_argnums=(3, 4, 5, 6))
def _segment_sum_sc(xt, src2, dst2, n, n_pad, dh, nb):
    mesh = plsc.VectorSubcoreMesh(core_axis_name="c", subcore_axis_name="s")
    rpt = n_pad // NS
    rx = n // NS
    g = nb // NS
    xtra = nb - g * NS
    hg = g // 2
    ntri = hg // 3
    rem = hg - 3 * ntri

    @functools.partial(
        pl.kernel,
        out_type=jax.ShapeDtypeStruct((NC, n_pad, dh), jnp.float32),
        mesh=mesh,
        compiler_params=pltpu.CompilerParams(use_tc_tiling_on_sc=False),
        scratch_types=[
            pltpu.VMEM((hg + 1, CH), jnp.int32),
            pltpu.VMEM((hg + 1, CH), jnp.int32),
            pltpu.VMEM((CH, dh), jnp.float32),
            pltpu.VMEM((CH, dh), jnp.float32),
            pltpu.VMEM((CH, dh), jnp.float32),
            pltpu.VMEM_SHARED((n_pad, dh), jnp.float32),
            pltpu.VMEM_SHARED((n_pad, dh), jnp.float32),
            pltpu.SemaphoreType.DMA,
            pltpu.SemaphoreType.DMA,
            pltpu.SemaphoreType.DMA,
            pltpu.SemaphoreType.DMA,
        ],
    )
    def scatter_kernel(xt_hbm, src_hbm, dst_hbm, zer_hbm, out_hbm,
                       idx_s, idx_d, r0, r1, r2, xsh, acc,
                       sem0, sem1, sem2, isem):
        c = lax.axis_index("c")
        s = lax.axis_index("s")
        xh = xt_hbm.at[c]

        dx = pltpu.async_copy(xh.at[pl.ds(s * rx, rx)],
                              xsh.at[pl.ds(s * rx, rx)], isem)
        dz = pltpu.async_copy(zer_hbm, acc.at[pl.ds(s * rpt, rpt)], isem)
        d0 = pltpu.async_copy(src_hbm.at[pl.ds(s * g, hg)],
                              idx_s.at[pl.ds(0, hg)], isem)
        d1 = pltpu.async_copy(dst_hbm.at[pl.ds(s * g, hg)],
                              idx_d.at[pl.ds(0, hg)], isem)
        dx.wait()
        dz.wait()
        d0.wait()
        d1.wait()
        plsc.subcore_barrier()

        bufs = (r0, r1, r2)
        sems = (sem0, sem1, sem2)

        def gather(k, i):
            src = xsh if i < 2 else xh
            pltpu.async_copy(src.at[idx_s.at[k]], bufs[i], sems[i])

        def wait_scatter(k, i):
            src = xsh if i < 2 else xh
            pltpu.make_async_copy(src.at[idx_s.at[k]], bufs[i], sems[i]).wait()
            pltpu.sync_copy(bufs[i], acc.at[idx_d.at[k]], add=True)

        for h in range(2):
            if h == 1:
                pltpu.sync_copy(src_hbm.at[pl.ds(s * g + hg, hg)],
                                idx_s.at[pl.ds(0, hg)])
                pltpu.sync_copy(dst_hbm.at[pl.ds(s * g + hg, hg)],
                                idx_d.at[pl.ds(0, hg)])

                @pl.when(s < xtra)
                def _():
                    pltpu.sync_copy(src_hbm.at[g * NS + s], idx_s.at[hg])
                    pltpu.sync_copy(dst_hbm.at[g * NS + s], idx_d.at[hg])

            gather(0, 0)
            gather(1, 1)
            gather(2, 2)

            def body(t, carry):
                k0 = 3 * t
                for i in range(3):
                    wait_scatter(k0 + i, i)

                    @pl.when(k0 + i + 3 < hg)
                    def _():
                        gather(k0 + i + 3, i)

                return carry

            lax.fori_loop(0, ntri, body, 0)

            for i in range(rem):
                wait_scatter(3 * ntri + i, i)

            if h == 1:
                @pl.when(s < xtra)
                def _():
                    pltpu.sync_copy(xsh.at[idx_s.at[hg]], r0)
                    pltpu.sync_copy(r0, acc.at[idx_d.at[hg]], add=True)

        plsc.subcore_barrier()
        pltpu.sync_copy(acc.at[pl.ds(s * rpt, rpt)],
                        out_hbm.at[c].at[pl.ds(s * rpt, rpt)])

    zer = jnp.zeros((rpt, dh), jnp.float32)
    return scatter_kernel(xt, src2, dst2, zer)


def kernel(x, edge_index, mask):
    n, d = x.shape
    e = edge_index.shape[1]
    dh = d // NC
    n_pad = ((n + 8 * NS - 1) // (8 * NS)) * (8 * NS)
    if n_pad == n:
        n_pad += 8 * NS
    nb = e // CH
    xt = jnp.stack([x[:, :dh], x[:, dh:]], axis=0)
    src2 = edge_index[0].reshape(nb, CH)
    dst2 = edge_index[1].reshape(nb, CH)
    out2 = _segment_sum_sc(xt, src2, dst2, n, n_pad, dh, nb)
    return jnp.concatenate([out2[0, :n], out2[1, :n]], axis=1)


# speedup vs baseline: 1.0370x; 1.0370x over previous
import functools

import jax
import jax.numpy as jnp
from jax import lax
from jax.experimental import pallas as pl
from jax.experimental.pallas import tpu as pltpu
from jax.experimental.pallas import tpu_sc as plsc

NC = 2
NS = 16
CH = 128


@functools.partial(jax.jit, static_argnums=(3, 4, 5, 6))
def _segment_sum_sc(xt, src2, dst2, n, n_pad, dh, nb):
    mesh = plsc.VectorSubcoreMesh(core_axis_name="c", subcore_axis_name="s")
    rpt = n_pad // NS
    rx = n // NS
    g = nb // NS
    xtra = nb - g * NS
    hg = g // 2
    ntri = hg // 3
    rem = hg - 3 * ntri

    @functools.partial(
        pl.kernel,
        out_type=jax.ShapeDtypeStruct((NC, n_pad, dh), jnp.float32),
        mesh=mesh,
        compiler_params=pltpu.CompilerParams(use_tc_tiling_on_sc=False),
        scratch_types=[
            pltpu.VMEM((hg + 1, CH), jnp.int32),
            pltpu.VMEM((hg + 1, CH), jnp.int32),
            pltpu.VMEM((CH, dh), jnp.float32),
            pltpu.VMEM((CH, dh), jnp.float32),
            pltpu.VMEM((CH, dh), jnp.float32),
            pltpu.VMEM_SHARED((n_pad, dh), jnp.float32),
            pltpu.VMEM_SHARED((n_pad, dh), jnp.float32),
            pltpu.SemaphoreType.DMA,
            pltpu.SemaphoreType.DMA,
            pltpu.SemaphoreType.DMA,
            pltpu.SemaphoreType.DMA,
        ],
    )
    def scatter_kernel(xt_hbm, src_hbm, dst_hbm, zer_hbm, out_hbm,
                       idx_s, idx_d, r0, r1, r2, xsh, acc,
                       sem0, sem1, sem2, isem):
        c = lax.axis_index("c")
        s = lax.axis_index("s")
        xh = xt_hbm.at[c]

        dx = pltpu.async_copy(xh.at[pl.ds(s * rx, rx)],
                              xsh.at[pl.ds(s * rx, rx)], isem)
        dz = pltpu.async_copy(zer_hbm, acc.at[pl.ds(s * rpt, rpt)], isem)
        d0 = pltpu.async_copy(src_hbm.at[pl.ds(s * g, hg)],
                              idx_s.at[pl.ds(0, hg)], isem)
        d1 = pltpu.async_copy(dst_hbm.at[pl.ds(s * g, hg)],
                              idx_d.at[pl.ds(0, hg)], isem)
        dx.wait()
        dz.wait()
        d0.wait()
        d1.wait()
        plsc.subcore_barrier()

        bufs = (r0, r1, r2)
        sems = (sem0, sem1, sem2)

        def gather(k, i):
            pltpu.async_copy(xsh.at[idx_s.at[k]], bufs[i], sems[i])

        def wait_scatter(k, i):
            pltpu.make_async_copy(xsh.at[idx_s.at[k]], bufs[i], sems[i]).wait()
            pltpu.sync_copy(bufs[i], acc.at[idx_d.at[k]], add=True)

        for h in range(2):
            if h == 1:
                pltpu.sync_copy(src_hbm.at[pl.ds(s * g + hg, hg)],
                                idx_s.at[pl.ds(0, hg)])
                pltpu.sync_copy(dst_hbm.at[pl.ds(s * g + hg, hg)],
                                idx_d.at[pl.ds(0, hg)])

                @pl.when(s < xtra)
                def _():
                    pltpu.sync_copy(src_hbm.at[g * NS + s], idx_s.at[hg])
                    pltpu.sync_copy(dst_hbm.at[g * NS + s], idx_d.at[hg])

            gather(0, 0)
            gather(1, 1)
            gather(2, 2)

            def body(t, carry):
                k0 = 3 * t
                for i in range(3):
                    wait_scatter(k0 + i, i)

                    @pl.when(k0 + i + 3 < hg)
                    def _():
                        gather(k0 + i + 3, i)

                return carry

            lax.fori_loop(0, ntri, body, 0)

            for i in range(rem):
                wait_scatter(3 * ntri + i, i)

            if h == 1:
                @pl.when(s < xtra)
                def _():
                    pltpu.sync_copy(xsh.at[idx_s.at[hg]], r0)
                    pltpu.sync_copy(r0, acc.at[idx_d.at[hg]], add=True)

        plsc.subcore_barrier()
        pltpu.sync_copy(acc.at[pl.ds(s * rpt, rpt)],
                        out_hbm.at[c].at[pl.ds(s * rpt, rpt)])

    zer = jnp.zeros((rpt, dh), jnp.float32)
    return scatter_kernel(xt, src2, dst2, zer)


def kernel(x, edge_index, mask):
    n, d = x.shape
    e = edge_index.shape[1]
    dh = d // NC
    n_pad = ((n + 8 * NS - 1) // (8 * NS)) * (8 * NS)
    if n_pad == n:
        n_pad += 8 * NS
    nb = e // CH
    xt = jnp.stack([x[:, :dh], x[:, dh:]], axis=0)
    src2 = edge_index[0].reshape(nb, CH)
    dst2 = edge_index[1].reshape(nb, CH)
    out2 = _segment_sum_sc(xt, src2, dst2, n, n_pad, dh, nb)
    return jnp.concatenate([out2[0, :n], out2[1, :n]], axis=1)


# speedup vs baseline: 1.1015x; 1.0622x over previous
import functools

import jax
import jax.numpy as jnp
from jax import lax
from jax.experimental import pallas as pl
from jax.experimental.pallas import tpu as pltpu
from jax.experimental.pallas import tpu_sc as plsc

NC = 2
NS = 16
CH = 128


@functools.partial(jax.jit, static_argnums=(3, 4, 5, 6))
def _segment_sum_sc(x, src2, dst2, n, n_pad, dh, nb):
    mesh = plsc.VectorSubcoreMesh(core_axis_name="c", subcore_axis_name="s")
    rpt = n_pad // NS
    rx = n // NS
    g = nb // NS
    xtra = nb - g * NS
    hg = g // 2
    ntri = hg // 3
    rem = hg - 3 * ntri

    @functools.partial(
        pl.kernel,
        out_type=jax.ShapeDtypeStruct((NC, n_pad, dh), jnp.float32),
        mesh=mesh,
        compiler_params=pltpu.CompilerParams(use_tc_tiling_on_sc=False),
        scratch_types=[
            pltpu.VMEM((hg + 1, CH), jnp.int32),
            pltpu.VMEM((hg + 1, CH), jnp.int32),
            pltpu.VMEM((CH, dh), jnp.float32),
            pltpu.VMEM((CH, dh), jnp.float32),
            pltpu.VMEM((CH, dh), jnp.float32),
            pltpu.VMEM_SHARED((n_pad, dh), jnp.float32),
            pltpu.VMEM_SHARED((n_pad, dh), jnp.float32),
            pltpu.SemaphoreType.DMA,
            pltpu.SemaphoreType.DMA,
            pltpu.SemaphoreType.DMA,
            pltpu.SemaphoreType.DMA,
        ],
    )
    def scatter_kernel(x_hbm, src_hbm, dst_hbm, zer_hbm, out_hbm,
                       idx_s, idx_d, r0, r1, r2, xsh, acc,
                       sem0, sem1, sem2, isem):
        c = lax.axis_index("c")
        s = lax.axis_index("s")

        dx = pltpu.async_copy(x_hbm.at[pl.ds(s * rx, rx), pl.ds(c * dh, dh)],
                              xsh.at[pl.ds(s * rx, rx)], isem)
        dz = pltpu.async_copy(zer_hbm, acc.at[pl.ds(s * rpt, rpt)], isem)
        dx.wait()
        dz.wait()
        plsc.subcore_barrier()

        bufs = (r0, r1, r2)
        sems = (sem0, sem1, sem2)

        def gather(k, i):
            pltpu.async_copy(xsh.at[idx_s.at[k]], bufs[i], sems[i])

        def wait_scatter(k, i):
            pltpu.make_async_copy(xsh.at[idx_s.at[k]], bufs[i], sems[i]).wait()
            pltpu.sync_copy(bufs[i], acc.at[idx_d.at[k]], add=True)

        for h in range(2):
            pltpu.sync_copy(src_hbm.at[pl.ds(s * g + h * hg, hg)],
                            idx_s.at[pl.ds(0, hg)])
            pltpu.sync_copy(dst_hbm.at[pl.ds(s * g + h * hg, hg)],
                            idx_d.at[pl.ds(0, hg)])
            if h == 1:
                @pl.when(s < xtra)
                def _():
                    pltpu.sync_copy(src_hbm.at[g * NS + s], idx_s.at[hg])
                    pltpu.sync_copy(dst_hbm.at[g * NS + s], idx_d.at[hg])

            gather(0, 0)
            gather(1, 1)
            gather(2, 2)

            def body(t, carry):
                k0 = 3 * t
                for i in range(3):
                    wait_scatter(k0 + i, i)

                    @pl.when(k0 + i + 3 < hg)
                    def _():
                        gather(k0 + i + 3, i)

                return carry

            lax.fori_loop(0, ntri, body, 0)

            for i in range(rem):
                wait_scatter(3 * ntri + i, i)

            if h == 1:
                @pl.when(s < xtra)
                def _():
                    pltpu.sync_copy(xsh.at[idx_s.at[hg]], r0)
                    pltpu.sync_copy(r0, acc.at[idx_d.at[hg]], add=True)

        plsc.subcore_barrier()
        pltpu.sync_copy(acc.at[pl.ds(s * rpt, rpt)],
                        out_hbm.at[c].at[pl.ds(s * rpt, rpt)])

    zer = jnp.zeros((rpt, dh), jnp.float32)
    return scatter_kernel(x, src2, dst2, zer)


def kernel(x, edge_index, mask):
    n, d = x.shape
    e = edge_index.shape[1]
    dh = d // NC
    n_pad = ((n + 8 * NS - 1) // (8 * NS)) * (8 * NS)
    if n_pad == n:
        n_pad += 8 * NS
    nb = e // CH
    src2 = edge_index[0].reshape(nb, CH)
    dst2 = edge_index[1].reshape(nb, CH)
    out2 = _segment_sum_sc(x, src2, dst2, n, n_pad, dh, nb)
    return jnp.concatenate([out2[0, :n], out2[1, :n]], axis=1)
